# baseline (speedup 1.0000x reference)
import jax
import jax.numpy as jnp
from jax.experimental import pallas as pl


def _gcn_body(x_ref, w1_ref, b1_ref, w2_ref, b2_ref, w3_ref, b3_ref, o_ref):
    x = x_ref[0]
    n = x.shape[0]
    xm = jnp.sum(x, axis=0, keepdims=True) * (1.0 / n)
    y = jnp.dot(xm, w1_ref[...], preferred_element_type=jnp.float32)
    y = jnp.maximum(y + b1_ref[...], 0.0)
    y = jnp.dot(y, w2_ref[...], preferred_element_type=jnp.float32)
    y = jnp.maximum(y + b2_ref[...], 0.0)
    y = jnp.dot(y, w3_ref[...], preferred_element_type=jnp.float32)
    y = jnp.maximum(y + b3_ref[...], 0.0)
    o_ref[...] = jnp.broadcast_to(y[:, None, :], o_ref.shape)


def kernel(x, W1, b1, W2, b2, W3, b3):
    B, N, C = x.shape
    hid = W1.shape[0]
    out_dim = W3.shape[0]
    out_rows = 128
    const = lambda i: (0, 0)
    return pl.pallas_call(
        _gcn_body,
        grid=(B,),
        in_specs=[
            pl.BlockSpec((1, N, C), lambda i: (i, 0, 0)),
            pl.BlockSpec((C, hid), const),
            pl.BlockSpec((1, hid), const),
            pl.BlockSpec((hid, hid), const),
            pl.BlockSpec((1, hid), const),
            pl.BlockSpec((hid, out_dim), const),
            pl.BlockSpec((1, out_dim), const),
        ],
        out_specs=pl.BlockSpec((1, out_rows, out_dim), lambda i: (i, 0, 0)),
        out_shape=jax.ShapeDtypeStruct((B, out_rows, out_dim), x.dtype),
    )(
        x,
        W1.T, b1.reshape(1, -1),
        W2.T, b2.reshape(1, -1),
        W3.T, b3.reshape(1, -1),
    )


# speedup vs baseline: 2.1241x; 2.1241x over previous
import jax
import jax.numpy as jnp
from jax import lax
from jax.experimental import pallas as pl

_DN_T = (((1,), (1,)), ((), ()))


def _gcn_body(x_ref, w1_ref, b1_ref, w2_ref, b2_ref, w3_ref, b3_ref, o_ref):
    x = x_ref[...]
    n = x.shape[1]
    xm = jnp.sum(x, axis=1) * (1.0 / n)
    y = lax.dot_general(xm, w1_ref[...], _DN_T, preferred_element_type=jnp.float32)
    y = jnp.maximum(y + b1_ref[...], 0.0)
    y = lax.dot_general(y, w2_ref[...], _DN_T, preferred_element_type=jnp.float32)
    y = jnp.maximum(y + b2_ref[...], 0.0)
    y = lax.dot_general(y, w3_ref[...], _DN_T, preferred_element_type=jnp.float32)
    y = jnp.maximum(y + b3_ref[...], 0.0)
    o_ref[...] = jnp.broadcast_to(y[:, None, :], o_ref.shape)


def kernel(x, W1, b1, W2, b2, W3, b3):
    B, N, C = x.shape
    out_dim = W3.shape[0]
    out_rows = 128
    return pl.pallas_call(
        _gcn_body,
        out_shape=jax.ShapeDtypeStruct((B, out_rows, out_dim), x.dtype),
    )(
        x,
        W1, b1.reshape(1, -1),
        W2, b2.reshape(1, -1),
        W3, b3.reshape(1, -1),
    )
